# Initial kernel scaffold; baseline (speedup 1.0000x reference)
import jax
import jax.numpy as jnp
from jax.experimental import pallas as pl


def kernel(x, W):
    raise NotImplementedError("write your pallas kernel here")


# speedup vs baseline: 1.4695x; 1.4695x over previous
import functools

import jax
import jax.numpy as jnp
from jax.experimental import pallas as pl
from jax.experimental.pallas import tpu as pltpu

NUM_EMBED_K = 100000
EMB_DIM_K = 768
TOPK_K = 8
N_TOKENS_K = 64

BLOCK_ROWS = 2048


def _topk_kernel(x_ref, w_ref, vals_ref, idx_ref, run_v_ref, run_i_ref):
    i = pl.program_id(0)
    nsteps = pl.num_programs(0)

    @pl.when(i == 0)
    def _init():
        run_v_ref[...] = jnp.full(run_v_ref.shape, -jnp.inf, jnp.float32)
        run_i_ref[...] = jnp.zeros(run_i_ref.shape, jnp.int32)

    x = x_ref[...]
    w = w_ref[...]
    logits = jax.lax.dot_general(
        x, w, (((1,), (1,)), ((), ())), preferred_element_type=jnp.float32
    )

    b = logits.shape[1]
    base = i * b
    cols = base + jax.lax.broadcasted_iota(jnp.int32, logits.shape, 1)
    logits = jnp.where(cols < NUM_EMBED_K, logits, -jnp.inf)

    ext_v = jnp.concatenate([logits, run_v_ref[...]], axis=1)
    ext_i = jnp.concatenate([cols, run_i_ref[...]], axis=1)

    BIG = jnp.int32(2**30)
    out_vs = []
    out_is = []
    for _ in range(TOPK_K):
        m = jnp.max(ext_v, axis=1, keepdims=True)
        gi = jnp.min(jnp.where(ext_v == m, ext_i, BIG), axis=1, keepdims=True)
        out_vs.append(m)
        out_is.append(gi)
        ext_v = jnp.where(ext_i == gi, -jnp.inf, ext_v)

    new_v = jnp.concatenate(out_vs, axis=1)
    new_i = jnp.concatenate(out_is, axis=1)
    run_v_ref[:, 0:TOPK_K] = new_v
    run_i_ref[:, 0:TOPK_K] = new_i

    @pl.when(i == nsteps - 1)
    def _emit():
        vals_ref[...] = new_v
        idx_ref[...] = new_i


@jax.jit
def kernel(x, W):
    nsteps = pl.cdiv(NUM_EMBED_K, BLOCK_ROWS)
    vals, idx = pl.pallas_call(
        _topk_kernel,
        grid=(nsteps,),
        in_specs=[
            pl.BlockSpec((N_TOKENS_K, EMB_DIM_K), lambda i: (0, 0)),
            pl.BlockSpec((BLOCK_ROWS, EMB_DIM_K), lambda i: (i, 0)),
        ],
        out_specs=[
            pl.BlockSpec((N_TOKENS_K, TOPK_K), lambda i: (0, 0)),
            pl.BlockSpec((N_TOKENS_K, TOPK_K), lambda i: (0, 0)),
        ],
        out_shape=[
            jax.ShapeDtypeStruct((N_TOKENS_K, TOPK_K), jnp.float32),
            jax.ShapeDtypeStruct((N_TOKENS_K, TOPK_K), jnp.int32),
        ],
        scratch_shapes=[
            pltpu.VMEM((N_TOKENS_K, 128), jnp.float32),
            pltpu.VMEM((N_TOKENS_K, 128), jnp.int32),
        ],
    )(x, W)
    return (vals, idx)
